# Initial kernel scaffold; baseline (speedup 1.0000x reference)
import functools

import jax
import jax.numpy as jnp
from jax import lax
from jax.experimental import pallas as pl
from jax.experimental.pallas import tpu as pltpu
from jax.experimental.pallas import tpu_sc as plsc

_N_USERS = 10000
_N_DISHES = 10000
_E = 320000
_D = 128

_NS = 16
_CHUNK = 128
_CPT = 157
_E_PAD = _NS * _CPT * _CHUNK
_ACC_ROWS = 10240
_TRASH = 10000
_ZPT = _ACC_ROWS // _NS
_OPT = _N_USERS // _NS


def _proj2_body(x_ref, wa_ref, ba_ref, wb_ref, bb_ref, a_ref, b_ref):
    x = x_ref[...]
    a_ref[...] = jnp.maximum(
        jnp.dot(x, wa_ref[...], preferred_element_type=jnp.float32)
        + ba_ref[...], 0.0)
    b_ref[...] = jnp.maximum(
        jnp.dot(x, wb_ref[...], preferred_element_type=jnp.float32)
        + bb_ref[...], 0.0)


def _proj2(x, wa, ba, wb, bb, bn=1000):
    n = x.shape[0]
    xspec = pl.BlockSpec((bn, _D), lambda i: (i, 0))
    wspec = pl.BlockSpec((_D, _D), lambda i: (0, 0))
    bspec = pl.BlockSpec((1, _D), lambda i: (0, 0))
    return pl.pallas_call(
        _proj2_body,
        grid=(n // bn,),
        in_specs=[xspec, wspec, bspec, wspec, bspec],
        out_specs=[xspec, xspec],
        out_shape=[jax.ShapeDtypeStruct((n, _D), jnp.float32)] * 2,
    )(x, wa, ba.reshape(1, _D), wb, bb.reshape(1, _D))


def _post_body(agg_ref, init_ref, wu_ref, bu_ref, wp_ref, bp_ref, out_ref):
    h = jnp.maximum(
        jnp.dot(agg_ref[...] + init_ref[...], wu_ref[...],
                preferred_element_type=jnp.float32) + bu_ref[...], 0.0)
    out_ref[...] = (jnp.dot(h, wp_ref[...],
                            preferred_element_type=jnp.float32) + bp_ref[...])


def _post(agg, init, wu, bu, wp, bp, bn=1000):
    n = agg.shape[0]
    xspec = pl.BlockSpec((bn, _D), lambda i: (i, 0))
    wspec = pl.BlockSpec((_D, _D), lambda i: (0, 0))
    bspec = pl.BlockSpec((1, _D), lambda i: (0, 0))
    return pl.pallas_call(
        _post_body,
        grid=(n // bn,),
        in_specs=[xspec, xspec, wspec, bspec, wspec, bspec],
        out_specs=xspec,
        out_shape=jax.ShapeDtypeStruct((n, _D), jnp.float32),
    )(agg, init, wu, bu.reshape(1, _D), wp, bp.reshape(1, _D))


def _sc_agg(g0, s0, g1, s1, table_u, table_d):
    mesh = plsc.VectorSubcoreMesh(core_axis_name="c", subcore_axis_name="s")

    @functools.partial(
        pl.kernel,
        out_type=[jax.ShapeDtypeStruct((_N_USERS, _D), jnp.float32),
                  jax.ShapeDtypeStruct((_N_DISHES, _D), jnp.float32)],
        mesh=mesh,
        scratch_types=[
            pltpu.VMEM((_CPT, _CHUNK), jnp.int32),
            pltpu.VMEM((_CPT, _CHUNK), jnp.int32),
            pltpu.VMEM((_CHUNK, _D), jnp.float32),
            pltpu.VMEM_SHARED((_ACC_ROWS, _D), jnp.float32),
            pltpu.SemaphoreType.DMA,
        ],
    )
    def k(g0_h, s0_h, g1_h, s1_h, tu_h, td_h, uout_h, dout_h,
          gidx, sidx, rows, acc, sem):
        core = lax.axis_index("c")
        tile = lax.axis_index("s")

        zero = jnp.zeros((16,), jnp.float32)

        def zrow(i, c):
            rows[i // 8, pl.ds((i % 8) * 16, 16)] = zero
            return c
        lax.fori_loop(0, _CHUNK * 8, zrow, 0)
        for z in range(_ZPT // _CHUNK):
            pltpu.sync_copy(rows, acc.at[pl.ds(tile * _ZPT + z * _CHUNK,
                                               _CHUNK)])
        plsc.subcore_barrier()

        def side(g_h, s_h, t_h, o_h):
            pltpu.sync_copy(g_h.at[pl.ds(tile * _CPT, _CPT)], gidx)
            pltpu.sync_copy(s_h.at[pl.ds(tile * _CPT, _CPT)], sidx)

            def body(j, c):
                pltpu.async_copy(t_h.at[gidx.at[j]], rows, sem).wait()
                pltpu.sync_copy(rows, acc.at[sidx.at[j]], add=True)
                return c
            lax.fori_loop(0, _CPT, body, 0)
            plsc.subcore_barrier()
            pltpu.sync_copy(acc.at[pl.ds(tile * _OPT, _OPT)],
                            o_h.at[pl.ds(tile * _OPT, _OPT)])

        @pl.when(core == 0)
        def _user_side():
            side(g0_h, s0_h, tu_h, uout_h)

        @pl.when(core == 1)
        def _dish_side():
            side(g1_h, s1_h, td_h, dout_h)

    return k(g0, s0, g1, s1, table_u, table_d)


def kernel(user_x, dish_x, edge_index,
           W_ui, b_ui, W_di, b_di,
           W_um, b_um, W_uu, b_uu, W_up, b_up,
           W_dm, b_dm, W_du, b_du, W_dp, b_dp):
    e_user = edge_index[0]
    e_dish = edge_index[1]
    pad = _E_PAD - _E
    zpad = jnp.zeros((pad,), jnp.int32)
    tpad = jnp.full((pad,), _TRASH, jnp.int32)
    g0 = jnp.concatenate([e_dish, zpad]).reshape(_NS * _CPT, _CHUNK)
    s0 = jnp.concatenate([e_user, tpad]).reshape(_NS * _CPT, _CHUNK)
    g1 = jnp.concatenate([e_user, zpad]).reshape(_NS * _CPT, _CHUNK)
    s1 = jnp.concatenate([e_dish, tpad]).reshape(_NS * _CPT, _CHUNK)

    user_initial, table_d = _proj2(user_x, W_ui, b_ui, W_dm, b_dm)
    dish_initial, table_u = _proj2(dish_x, W_di, b_di, W_um, b_um)

    user_agg, dish_agg = _sc_agg(g0, s0, g1, s1, table_u, table_d)

    user_emb = _post(user_agg, user_initial, W_uu, b_uu, W_up, b_up)
    dish_emb = _post(dish_agg, dish_initial, W_du, b_du, W_dp, b_dp)
    return (user_emb, dish_emb)


# speedup vs baseline: 3.9343x; 3.9343x over previous
import functools

import jax
import jax.numpy as jnp
from jax import lax
from jax.experimental import pallas as pl
from jax.experimental.pallas import tpu as pltpu
from jax.experimental.pallas import tpu_sc as plsc

_N_USERS = 10000
_N_DISHES = 10000
_E = 320000
_D = 128

_NS = 16
_CHUNK = 128
_CPT = 160
_E_PAD = _NS * _CPT * _CHUNK
_ACC_ROWS = 10240
_TRASH = 10239
_IB = 16
_ZPT = _ACC_ROWS // _NS
_OPT = 632
_OUT_PAD = _NS * _OPT


def _proj2_body(x_ref, wa_ref, ba_ref, wb_ref, bb_ref, a_ref, b_ref):
    x = x_ref[...]
    a_ref[...] = jnp.maximum(
        jnp.dot(x, wa_ref[...], preferred_element_type=jnp.float32)
        + ba_ref[...], 0.0)
    b_ref[...] = jnp.maximum(
        jnp.dot(x, wb_ref[...], preferred_element_type=jnp.float32)
        + bb_ref[...], 0.0)


def _proj2(x, wa, ba, wb, bb, bn=1000):
    n = x.shape[0]
    xspec = pl.BlockSpec((bn, _D), lambda i: (i, 0))
    wspec = pl.BlockSpec((_D, _D), lambda i: (0, 0))
    bspec = pl.BlockSpec((1, _D), lambda i: (0, 0))
    return pl.pallas_call(
        _proj2_body,
        grid=(n // bn,),
        in_specs=[xspec, wspec, bspec, wspec, bspec],
        out_specs=[xspec, xspec],
        out_shape=[jax.ShapeDtypeStruct((n, _D), jnp.float32)] * 2,
    )(x, wa, ba.reshape(1, _D), wb, bb.reshape(1, _D))


def _post_body(agg_ref, init_ref, wu_ref, bu_ref, wp_ref, bp_ref, out_ref):
    h = jnp.maximum(
        jnp.dot(agg_ref[...] + init_ref[...], wu_ref[...],
                preferred_element_type=jnp.float32) + bu_ref[...], 0.0)
    out_ref[...] = (jnp.dot(h, wp_ref[...],
                            preferred_element_type=jnp.float32) + bp_ref[...])


def _post(agg, init, wu, bu, wp, bp, bn=1000):
    n = agg.shape[0]
    xspec = pl.BlockSpec((bn, _D), lambda i: (i, 0))
    wspec = pl.BlockSpec((_D, _D), lambda i: (0, 0))
    bspec = pl.BlockSpec((1, _D), lambda i: (0, 0))
    return pl.pallas_call(
        _post_body,
        grid=(n // bn,),
        in_specs=[xspec, xspec, wspec, bspec, wspec, bspec],
        out_specs=xspec,
        out_shape=jax.ShapeDtypeStruct((n, _D), jnp.float32),
    )(agg, init, wu, bu.reshape(1, _D), wp, bp.reshape(1, _D))


def _sc_agg(g0, s0, g1, s1, table_u, table_d):
    mesh = plsc.VectorSubcoreMesh(core_axis_name="c", subcore_axis_name="s")

    @functools.partial(
        pl.kernel,
        out_type=[jax.ShapeDtypeStruct((_OUT_PAD, _D), jnp.float32),
                  jax.ShapeDtypeStruct((_OUT_PAD, _D), jnp.float32)],
        mesh=mesh,
        scratch_types=[
            pltpu.VMEM((_IB, _CHUNK), jnp.int32),
            pltpu.VMEM((_IB, _CHUNK), jnp.int32),
            pltpu.VMEM((_CHUNK, _D), jnp.float32),
            pltpu.VMEM_SHARED((_ACC_ROWS, _D), jnp.float32),
            pltpu.SemaphoreType.DMA,
        ],
    )
    def k(g0_h, s0_h, g1_h, s1_h, tu_h, td_h, uout_h, dout_h,
          gidx, sidx, rows, acc, sem):
        core = lax.axis_index("c")
        tile = lax.axis_index("s")

        zero = jnp.zeros((16,), jnp.float32)

        def zrow(i, c):
            rows[i // 8, pl.ds((i % 8) * 16, 16)] = zero
            return c
        lax.fori_loop(0, _CHUNK * 8, zrow, 0)
        for z in range(_ZPT // _CHUNK):
            pltpu.sync_copy(rows, acc.at[pl.ds(tile * _ZPT + z * _CHUNK,
                                               _CHUNK)])
        plsc.subcore_barrier()

        def side(g_h, s_h, t_h, o_h):
            def blk(b, c):
                base = tile * _CPT + b * _IB
                pltpu.sync_copy(g_h.at[pl.ds(base, _IB)], gidx)
                pltpu.sync_copy(s_h.at[pl.ds(base, _IB)], sidx)

                def body(j, c2):
                    pltpu.async_copy(t_h.at[gidx.at[j]], rows, sem).wait()
                    pltpu.sync_copy(rows, acc.at[sidx.at[j]], add=True)
                    return c2
                lax.fori_loop(0, _IB, body, 0)
                return c
            lax.fori_loop(0, _CPT // _IB, blk, 0)
            plsc.subcore_barrier()
            pltpu.sync_copy(acc.at[pl.ds(tile * _OPT, _OPT)],
                            o_h.at[pl.ds(tile * _OPT, _OPT)])

        @pl.when(core == 0)
        def _user_side():
            side(g0_h, s0_h, tu_h, uout_h)

        @pl.when(core == 1)
        def _dish_side():
            side(g1_h, s1_h, td_h, dout_h)

    return k(g0, s0, g1, s1, table_u, table_d)


def kernel(user_x, dish_x, edge_index,
           W_ui, b_ui, W_di, b_di,
           W_um, b_um, W_uu, b_uu, W_up, b_up,
           W_dm, b_dm, W_du, b_du, W_dp, b_dp):
    e_user = edge_index[0]
    e_dish = edge_index[1]
    pad = _E_PAD - _E
    zpad = jnp.zeros((pad,), jnp.int32)
    tpad = jnp.full((pad,), _TRASH, jnp.int32)
    g0 = jnp.concatenate([e_dish, zpad]).reshape(_NS * _CPT, _CHUNK)
    s0 = jnp.concatenate([e_user, tpad]).reshape(_NS * _CPT, _CHUNK)
    g1 = jnp.concatenate([e_user, zpad]).reshape(_NS * _CPT, _CHUNK)
    s1 = jnp.concatenate([e_dish, tpad]).reshape(_NS * _CPT, _CHUNK)

    user_initial, table_d = _proj2(user_x, W_ui, b_ui, W_dm, b_dm)
    dish_initial, table_u = _proj2(dish_x, W_di, b_di, W_um, b_um)

    user_agg, dish_agg = _sc_agg(g0, s0, g1, s1, table_u, table_d)
    user_agg = user_agg[:_N_USERS]
    dish_agg = dish_agg[:_N_DISHES]

    user_emb = _post(user_agg, user_initial, W_uu, b_uu, W_up, b_up)
    dish_emb = _post(dish_agg, dish_initial, W_du, b_du, W_dp, b_dp)
    return (user_emb, dish_emb)


# speedup vs baseline: 4.3423x; 1.1037x over previous
import functools

import jax
import jax.numpy as jnp
from jax import lax
from jax.experimental import pallas as pl
from jax.experimental.pallas import tpu as pltpu
from jax.experimental.pallas import tpu_sc as plsc

_N_USERS = 10000
_N_DISHES = 10000
_E = 320000
_D = 128

_NS = 16
_CHUNK = 128
_CPT = 160
_E_PAD = _NS * _CPT * _CHUNK
_ACC_ROWS = 10240
_TRASH = 10239
_IB = 16
_ZPT = _ACC_ROWS // _NS
_OPT = 632
_OUT_PAD = _NS * _OPT


def _proj2_body(x_ref, wa_ref, ba_ref, wb_ref, bb_ref, a_ref, b_ref):
    x = x_ref[...]
    a_ref[...] = jnp.maximum(
        jnp.dot(x, wa_ref[...], preferred_element_type=jnp.float32)
        + ba_ref[...], 0.0)
    b_ref[...] = jnp.maximum(
        jnp.dot(x, wb_ref[...], preferred_element_type=jnp.float32)
        + bb_ref[...], 0.0)


def _proj2(x, wa, ba, wb, bb, bn=1000):
    n = x.shape[0]
    xspec = pl.BlockSpec((bn, _D), lambda i: (i, 0))
    wspec = pl.BlockSpec((_D, _D), lambda i: (0, 0))
    bspec = pl.BlockSpec((1, _D), lambda i: (0, 0))
    return pl.pallas_call(
        _proj2_body,
        grid=(n // bn,),
        in_specs=[xspec, wspec, bspec, wspec, bspec],
        out_specs=[xspec, xspec],
        out_shape=[jax.ShapeDtypeStruct((n, _D), jnp.float32)] * 2,
    )(x, wa, ba.reshape(1, _D), wb, bb.reshape(1, _D))


def _post_body(agg_ref, init_ref, wu_ref, bu_ref, wp_ref, bp_ref, out_ref):
    h = jnp.maximum(
        jnp.dot(agg_ref[...] + init_ref[...], wu_ref[...],
                preferred_element_type=jnp.float32) + bu_ref[...], 0.0)
    out_ref[...] = (jnp.dot(h, wp_ref[...],
                            preferred_element_type=jnp.float32) + bp_ref[...])


def _post(agg, init, wu, bu, wp, bp, bn=1000):
    n = agg.shape[0]
    xspec = pl.BlockSpec((bn, _D), lambda i: (i, 0))
    wspec = pl.BlockSpec((_D, _D), lambda i: (0, 0))
    bspec = pl.BlockSpec((1, _D), lambda i: (0, 0))
    return pl.pallas_call(
        _post_body,
        grid=(n // bn,),
        in_specs=[xspec, xspec, wspec, bspec, wspec, bspec],
        out_specs=xspec,
        out_shape=jax.ShapeDtypeStruct((n, _D), jnp.float32),
    )(agg, init, wu, bu.reshape(1, _D), wp, bp.reshape(1, _D))


def _sc_agg(g0, s0, g1, s1, table_u, table_d):
    mesh = plsc.VectorSubcoreMesh(core_axis_name="c", subcore_axis_name="s")

    @functools.partial(
        pl.kernel,
        out_type=[jax.ShapeDtypeStruct((_OUT_PAD, _D), jnp.float32),
                  jax.ShapeDtypeStruct((_OUT_PAD, _D), jnp.float32)],
        mesh=mesh,
        scratch_types=[
            pltpu.VMEM((_IB, _CHUNK), jnp.int32),
            pltpu.VMEM((_IB, _CHUNK), jnp.int32),
            pltpu.VMEM((_CHUNK, _D), jnp.float32),
            pltpu.VMEM((_CHUNK, _D), jnp.float32),
            pltpu.VMEM_SHARED((_ACC_ROWS, _D), jnp.float32),
            pltpu.SemaphoreType.DMA,
            pltpu.SemaphoreType.DMA,
            pltpu.SemaphoreType.DMA,
            pltpu.SemaphoreType.DMA,
        ],
    )
    def k(g0_h, s0_h, g1_h, s1_h, tu_h, td_h, uout_h, dout_h,
          gidx, sidx, rows, rows1, acc, sem, gsem1, ssem0, ssem1):
        core = lax.axis_index("c")
        tile = lax.axis_index("s")

        zero = jnp.zeros((16,), jnp.float32)

        def zrow(i, c):
            rows[i // 8, pl.ds((i % 8) * 16, 16)] = zero
            return c
        lax.fori_loop(0, _CHUNK * 8, zrow, 0)
        for z in range(_ZPT // _CHUNK):
            pltpu.sync_copy(rows, acc.at[pl.ds(tile * _ZPT + z * _CHUNK,
                                               _CHUNK)])
        plsc.subcore_barrier()

        def side(g_h, s_h, t_h, o_h):
            rbufs = (rows, rows1)
            gsems = (sem, gsem1)
            ssems = (ssem0, ssem1)

            def blk(b, c):
                base = tile * _CPT + b * _IB
                pltpu.sync_copy(g_h.at[pl.ds(base, _IB)], gidx)
                pltpu.sync_copy(s_h.at[pl.ds(base, _IB)], sidx)
                gd = [None, None]
                sd = [None, None]
                gd[0] = pltpu.async_copy(t_h.at[gidx.at[0]], rbufs[0],
                                         gsems[0])
                for j in range(_IB):
                    p = j % 2
                    q = (j + 1) % 2
                    gd[p].wait()
                    sd[p] = pltpu.async_copy(rbufs[p], acc.at[sidx.at[j]],
                                             ssems[p], add=True)
                    if j + 1 < _IB:
                        if j >= 1:
                            sd[q].wait()
                        gd[q] = pltpu.async_copy(t_h.at[gidx.at[j + 1]],
                                                 rbufs[q], gsems[q])
                    else:
                        sd[q].wait()
                        sd[p].wait()
                return c
            lax.fori_loop(0, _CPT // _IB, blk, 0)
            plsc.subcore_barrier()
            pltpu.sync_copy(acc.at[pl.ds(tile * _OPT, _OPT)],
                            o_h.at[pl.ds(tile * _OPT, _OPT)])

        @pl.when(core == 0)
        def _user_side():
            side(g0_h, s0_h, tu_h, uout_h)

        @pl.when(core == 1)
        def _dish_side():
            side(g1_h, s1_h, td_h, dout_h)

    return k(g0, s0, g1, s1, table_u, table_d)


def kernel(user_x, dish_x, edge_index,
           W_ui, b_ui, W_di, b_di,
           W_um, b_um, W_uu, b_uu, W_up, b_up,
           W_dm, b_dm, W_du, b_du, W_dp, b_dp):
    e_user = edge_index[0]
    e_dish = edge_index[1]
    pad = _E_PAD - _E
    zpad = jnp.zeros((pad,), jnp.int32)
    tpad = jnp.full((pad,), _TRASH, jnp.int32)
    g0 = jnp.concatenate([e_dish, zpad]).reshape(_NS * _CPT, _CHUNK)
    s0 = jnp.concatenate([e_user, tpad]).reshape(_NS * _CPT, _CHUNK)
    g1 = jnp.concatenate([e_user, zpad]).reshape(_NS * _CPT, _CHUNK)
    s1 = jnp.concatenate([e_dish, tpad]).reshape(_NS * _CPT, _CHUNK)

    user_initial, table_d = _proj2(user_x, W_ui, b_ui, W_dm, b_dm)
    dish_initial, table_u = _proj2(dish_x, W_di, b_di, W_um, b_um)

    user_agg, dish_agg = _sc_agg(g0, s0, g1, s1, table_u, table_d)
    user_agg = user_agg[:_N_USERS]
    dish_agg = dish_agg[:_N_DISHES]

    user_emb = _post(user_agg, user_initial, W_uu, b_uu, W_up, b_up)
    dish_emb = _post(dish_agg, dish_initial, W_du, b_du, W_dp, b_dp)
    return (user_emb, dish_emb)
